# baseline (speedup 1.0000x reference)
import functools

import jax
import jax.numpy as jnp
from jax import lax
from jax.experimental import pallas as pl
from jax.experimental.pallas import tpu as pltpu
from jax.experimental.pallas import tpu_sc as plsc

NC = 2
NS = 16
NW = NC * NS
L = 16

CHUNK = 80


def _decoder_kernel(z_hbm, src_hbm, dst_hbm, out_hbm,
                    idx_s, idx_d, rows_s, rows_d, out_v, tr,
                    sem_is, sem_id, sem_s0, sem_d0, sem_s1, sem_d1):
    b = out_hbm.shape[0]
    w = z_hbm.shape[1]
    b_per_w = b // NW

    wid = lax.axis_index("s") * NC + lax.axis_index("c")
    base = wid * b_per_w
    n_chunks = b_per_w // CHUNK

    cp_is = pltpu.async_copy(src_hbm.at[pl.ds(base, b_per_w)], idx_s, sem_is)
    cp_id = pltpu.async_copy(dst_hbm.at[pl.ds(base, b_per_w)], idx_d, sem_id)
    cp_is.wait()
    cp_id.wait()

    sems = ((sem_s0, sem_d0), (sem_s1, sem_d1))

    def issue(ci, buf):
        ss, sd = sems[buf]
        cs = pltpu.async_copy(
            z_hbm.at[idx_s.at[pl.ds(ci * CHUNK, CHUNK)]],
            rows_s.at[buf], ss)
        cd = pltpu.async_copy(
            z_hbm.at[idx_d.at[pl.ds(ci * CHUNK, CHUNK)]],
            rows_d.at[buf], sd)
        return cs, cd

    def wait(buf):
        ss, sd = sems[buf]
        pltpu.make_async_copy(z_hbm.at[idx_s.at[pl.ds(0, CHUNK)]],
                              rows_s.at[buf], ss).wait()
        pltpu.make_async_copy(z_hbm.at[idx_d.at[pl.ds(0, CHUNK)]],
                              rows_d.at[buf], sd).wait()

    lane = lax.iota(jnp.int32, L)
    m_last = lane == (L - 1)

    def compute(ci, buf):
        rs = rows_s.at[buf]
        rd = rows_d.at[buf]

        def edge_body(em, carry2):
            acc = jnp.zeros((L,), jnp.float32)
            for k in range(w // L):
                vs = plsc.bitcast(rs[em, pl.ds(k * L, L)], jnp.bfloat16)
                vd = plsc.bitcast(rd[em, pl.ds(k * L, L)], jnp.bfloat16)
                p = vs * vd
                u0, u1 = plsc.unpack(p, format=plsc.PackFormat.INTERLEAVED)
                acc = acc + u0 + u1
            cs = plsc.cumsum(acc)
            plsc.store_scatter(
                out_v, [jnp.full((L,), ci * CHUNK + em, jnp.int32)], cs,
                mask=m_last)
            return carry2

        lax.fori_loop(0, CHUNK, edge_body, 0, unroll=8)

    issue(0, 0)

    def pair_body(h, carry):
        i = h * 2
        issue(i + 1, 1)
        wait(0)
        compute(i, 0)
        issue(i + 2, 0)
        wait(1)
        compute(i + 1, 1)
        return carry

    lax.fori_loop(0, (n_chunks - 1) // 2, pair_body, 0)
    wait(0)
    compute(n_chunks - 1, 0)

    pltpu.sync_copy(out_v, out_hbm.at[pl.ds(base, b_per_w)])


def kernel(z, source, destination):
    b = source.shape[0]
    d = z.shape[1]
    b_per_w = b // NW
    u = lax.bitcast_convert_type(z, jnp.uint32)
    r = (u + jnp.uint32(0x7FFF) + ((u >> 16) & jnp.uint32(1))) >> 16
    z = lax.bitcast_convert_type(r[:, :d // 2] | (r[:, d // 2:] << 16),
                                 jnp.int32)
    source = source.astype(jnp.int32)
    destination = destination.astype(jnp.int32)

    run = functools.partial(
        pl.kernel,
        mesh=plsc.VectorSubcoreMesh(core_axis_name="c", subcore_axis_name="s"),
        compiler_params=pltpu.CompilerParams(
            needs_layout_passes=False, use_tc_tiling_on_sc=False),
        out_type=jax.ShapeDtypeStruct((b,), jnp.float32),
        scratch_types=[
            pltpu.VMEM((b_per_w,), jnp.int32),
            pltpu.VMEM((b_per_w,), jnp.int32),
            pltpu.VMEM((2, CHUNK, d // 2), jnp.int32),
            pltpu.VMEM((2, CHUNK, d // 2), jnp.int32),
            pltpu.VMEM((b_per_w,), jnp.float32),
            pltpu.VMEM((L * L,), jnp.float32),
            pltpu.SemaphoreType.DMA,
            pltpu.SemaphoreType.DMA,
            pltpu.SemaphoreType.DMA,
            pltpu.SemaphoreType.DMA,
            pltpu.SemaphoreType.DMA,
            pltpu.SemaphoreType.DMA,
        ],
    )(_decoder_kernel)
    return run(z, source, destination)


# speedup vs baseline: 2.0336x; 2.0336x over previous
import functools

import jax
import jax.numpy as jnp
from jax import lax
from jax.experimental import pallas as pl
from jax.experimental.pallas import tpu as pltpu
from jax.experimental.pallas import tpu_sc as plsc

NC = 2
NS = 16
NW = NC * NS
L = 16

CHUNK = 80


def _decoder_kernel(z_hbm, src_hbm, dst_hbm, out_hbm,
                    idx_s, idx_d, rows_s, rows_d, out_v, tr,
                    sem_is, sem_id, sem_s0, sem_d0, sem_s1, sem_d1):
    b = out_hbm.shape[0]
    w = z_hbm.shape[1]
    b_per_w = b // NW

    wid = lax.axis_index("s") * NC + lax.axis_index("c")
    base = wid * b_per_w
    n_chunks = b_per_w // CHUNK

    cp_is = pltpu.async_copy(src_hbm.at[pl.ds(base, b_per_w)], idx_s, sem_is)
    cp_id = pltpu.async_copy(dst_hbm.at[pl.ds(base, b_per_w)], idx_d, sem_id)
    cp_is.wait()
    cp_id.wait()

    sems = ((sem_s0, sem_d0), (sem_s1, sem_d1))

    def issue(ci, buf):
        ss, sd = sems[buf]
        cs = pltpu.async_copy(
            z_hbm.at[idx_s.at[pl.ds(ci * CHUNK, CHUNK)]],
            rows_s.at[buf], ss)
        cd = pltpu.async_copy(
            z_hbm.at[idx_d.at[pl.ds(ci * CHUNK, CHUNK)]],
            rows_d.at[buf], sd)
        return cs, cd

    def wait(buf):
        ss, sd = sems[buf]
        pltpu.make_async_copy(z_hbm.at[idx_s.at[pl.ds(0, CHUNK)]],
                              rows_s.at[buf], ss).wait()
        pltpu.make_async_copy(z_hbm.at[idx_d.at[pl.ds(0, CHUNK)]],
                              rows_d.at[buf], sd).wait()

    lane = lax.iota(jnp.int32, L)

    def compute(ci, buf):
        rs = rows_s.at[buf]
        rd = rows_d.at[buf]

        def group_wrap(g, carry2):
            def edge_body(em, tot):
                e = g * L + em
                acc = jnp.zeros((L,), jnp.float32)
                for k in range(0, w // L, 2):
                    vs0 = plsc.bitcast(rs[e, pl.ds(k * L, L)], jnp.bfloat16)
                    vd0 = plsc.bitcast(rd[e, pl.ds(k * L, L)], jnp.bfloat16)
                    vs1 = plsc.bitcast(rs[e, pl.ds((k + 1) * L, L)], jnp.bfloat16)
                    vd1 = plsc.bitcast(rd[e, pl.ds((k + 1) * L, L)], jnp.bfloat16)
                    p = vs0 * vd0 + vs1 * vd1
                    u0, u1 = plsc.unpack(p, format=plsc.PackFormat.INTERLEAVED)
                    acc = acc + u0 + u1
                val = jnp.sum(acc)
                return jnp.where(lane == em, jnp.full((L,), val, jnp.float32),
                                 tot)

            tot = lax.fori_loop(0, L, edge_body, jnp.zeros((L,), jnp.float32),
                                unroll=4)
            out_v[pl.ds(ci * CHUNK + g * L, L)] = tot
            return carry2

        lax.fori_loop(0, CHUNK // L, group_wrap, 0)

    issue(0, 0)

    def pair_body(h, carry):
        i = h * 2
        issue(i + 1, 1)
        wait(0)
        compute(i, 0)
        issue(i + 2, 0)
        wait(1)
        compute(i + 1, 1)
        return carry

    lax.fori_loop(0, (n_chunks - 1) // 2, pair_body, 0)
    wait(0)
    compute(n_chunks - 1, 0)

    pltpu.sync_copy(out_v, out_hbm.at[pl.ds(base, b_per_w)])


def kernel(z, source, destination):
    b = source.shape[0]
    d = z.shape[1]
    b_per_w = b // NW
    u = lax.bitcast_convert_type(z, jnp.uint32)
    r = (u + jnp.uint32(0x7FFF) + ((u >> 16) & jnp.uint32(1))) >> 16
    z = lax.bitcast_convert_type(r[:, :d // 2] | (r[:, d // 2:] << 16),
                                 jnp.int32)
    source = source.astype(jnp.int32)
    destination = destination.astype(jnp.int32)

    run = functools.partial(
        pl.kernel,
        mesh=plsc.VectorSubcoreMesh(core_axis_name="c", subcore_axis_name="s"),
        compiler_params=pltpu.CompilerParams(
            needs_layout_passes=False, use_tc_tiling_on_sc=False),
        out_type=jax.ShapeDtypeStruct((b,), jnp.float32),
        scratch_types=[
            pltpu.VMEM((b_per_w,), jnp.int32),
            pltpu.VMEM((b_per_w,), jnp.int32),
            pltpu.VMEM((2, CHUNK, d // 2), jnp.int32),
            pltpu.VMEM((2, CHUNK, d // 2), jnp.int32),
            pltpu.VMEM((b_per_w,), jnp.float32),
            pltpu.VMEM((L * L,), jnp.float32),
            pltpu.SemaphoreType.DMA,
            pltpu.SemaphoreType.DMA,
            pltpu.SemaphoreType.DMA,
            pltpu.SemaphoreType.DMA,
            pltpu.SemaphoreType.DMA,
            pltpu.SemaphoreType.DMA,
        ],
    )(_decoder_kernel)
    return run(z, source, destination)


# speedup vs baseline: 2.4333x; 1.1966x over previous
import functools

import jax
import jax.numpy as jnp
from jax import lax
from jax.experimental import pallas as pl
from jax.experimental.pallas import tpu as pltpu
from jax.experimental.pallas import tpu_sc as plsc

NC = 2
NS = 16
NW = NC * NS
L = 16

CHUNK = 200


def _decoder_kernel(z_hbm, src_hbm, dst_hbm, out_hbm,
                    idx_s, idx_d, rows_s, rows_d, out_v, tr,
                    sem_is, sem_id, sem_s0, sem_d0, sem_s1, sem_d1):
    b = out_hbm.shape[0]
    w = z_hbm.shape[1]
    b_per_w = b // NW

    wid = lax.axis_index("s") * NC + lax.axis_index("c")
    base = wid * b_per_w
    n_chunks = b_per_w // CHUNK

    cp_is = pltpu.async_copy(src_hbm.at[pl.ds(base, b_per_w)], idx_s, sem_is)
    cp_id = pltpu.async_copy(dst_hbm.at[pl.ds(base, b_per_w)], idx_d, sem_id)
    cp_is.wait()
    cp_id.wait()

    sems = ((sem_s0, sem_d0), (sem_s1, sem_d1))

    def issue(ci, buf):
        ss, sd = sems[buf]
        cs = pltpu.async_copy(
            z_hbm.at[idx_s.at[pl.ds(ci * CHUNK, CHUNK)]],
            rows_s.at[buf], ss)
        cd = pltpu.async_copy(
            z_hbm.at[idx_d.at[pl.ds(ci * CHUNK, CHUNK)]],
            rows_d.at[buf], sd)
        return cs, cd

    def wait(buf):
        ss, sd = sems[buf]
        pltpu.make_async_copy(z_hbm.at[idx_s.at[pl.ds(0, CHUNK)]],
                              rows_s.at[buf], ss).wait()
        pltpu.make_async_copy(z_hbm.at[idx_d.at[pl.ds(0, CHUNK)]],
                              rows_d.at[buf], sd).wait()

    lane = lax.iota(jnp.int32, L)

    def compute(ci, buf):
        rs = rows_s.at[buf]
        rd = rows_d.at[buf]

        def group_wrap(g, carry2):
            def edge_body(em, tot):
                e = g * L + em
                acc = jnp.zeros((L,), jnp.float32)
                for k in range(0, w // L, 2):
                    vs0 = plsc.bitcast(rs[e, pl.ds(k * L, L)], jnp.bfloat16)
                    vd0 = plsc.bitcast(rd[e, pl.ds(k * L, L)], jnp.bfloat16)
                    vs1 = plsc.bitcast(rs[e, pl.ds((k + 1) * L, L)], jnp.bfloat16)
                    vd1 = plsc.bitcast(rd[e, pl.ds((k + 1) * L, L)], jnp.bfloat16)
                    p = vs0 * vd0 + vs1 * vd1
                    u0, u1 = plsc.unpack(p, format=plsc.PackFormat.INTERLEAVED)
                    acc = acc + u0 + u1
                val = jnp.sum(acc)
                return jnp.where(lane == em, jnp.full((L,), val, jnp.float32),
                                 tot)

            tot = lax.fori_loop(0, L, edge_body, jnp.zeros((L,), jnp.float32),
                                unroll=4)
            out_v[pl.ds(ci * CHUNK + g * L, L)] = tot
            return carry2

        lax.fori_loop(0, CHUNK // L, group_wrap, 0)

    issue(0, 0)

    def pair_body(h, carry):
        i = h * 2
        issue(i + 1, 1)
        wait(0)
        compute(i, 0)
        issue(i + 2, 0)
        wait(1)
        compute(i + 1, 1)
        return carry

    lax.fori_loop(0, (n_chunks - 1) // 2, pair_body, 0)
    wait(0)
    compute(n_chunks - 1, 0)

    pltpu.sync_copy(out_v, out_hbm.at[pl.ds(base, b_per_w)])


def kernel(z, source, destination):
    b = source.shape[0]
    d = z.shape[1]
    b_per_w = b // NW
    u = lax.bitcast_convert_type(z, jnp.uint32)
    r = (u + jnp.uint32(0x7FFF) + ((u >> 16) & jnp.uint32(1))) >> 16
    z = lax.bitcast_convert_type(r[:, :d // 2] | (r[:, d // 2:] << 16),
                                 jnp.int32)
    source = source.astype(jnp.int32)
    destination = destination.astype(jnp.int32)

    run = functools.partial(
        pl.kernel,
        mesh=plsc.VectorSubcoreMesh(core_axis_name="c", subcore_axis_name="s"),
        compiler_params=pltpu.CompilerParams(
            needs_layout_passes=False, use_tc_tiling_on_sc=False),
        out_type=jax.ShapeDtypeStruct((b,), jnp.float32),
        scratch_types=[
            pltpu.VMEM((b_per_w,), jnp.int32),
            pltpu.VMEM((b_per_w,), jnp.int32),
            pltpu.VMEM((2, CHUNK, d // 2), jnp.int32),
            pltpu.VMEM((2, CHUNK, d // 2), jnp.int32),
            pltpu.VMEM((b_per_w,), jnp.float32),
            pltpu.VMEM((L * L,), jnp.float32),
            pltpu.SemaphoreType.DMA,
            pltpu.SemaphoreType.DMA,
            pltpu.SemaphoreType.DMA,
            pltpu.SemaphoreType.DMA,
            pltpu.SemaphoreType.DMA,
            pltpu.SemaphoreType.DMA,
        ],
    )(_decoder_kernel)
    return run(z, source, destination)


# speedup vs baseline: 2.4360x; 1.0011x over previous
import functools

import jax
import jax.numpy as jnp
from jax import lax
from jax.experimental import pallas as pl
from jax.experimental.pallas import tpu as pltpu
from jax.experimental.pallas import tpu_sc as plsc

NC = 2
NS = 16
NW = NC * NS
L = 16

CHUNK = 200


def _decoder_kernel(z_hbm, src_hbm, dst_hbm, out_hbm,
                    idx_s, idx_d, rows_s, rows_d, out_v, tr,
                    sem_is, sem_id, sem_s0, sem_d0, sem_s1, sem_d1):
    b = out_hbm.shape[0]
    w = z_hbm.shape[1]
    b_per_w = b // NW

    wid = lax.axis_index("s") * NC + lax.axis_index("c")
    base = wid * b_per_w
    n_chunks = b_per_w // CHUNK

    cp_is = pltpu.async_copy(src_hbm.at[pl.ds(base, b_per_w)], idx_s, sem_is)
    cp_id = pltpu.async_copy(dst_hbm.at[pl.ds(base, b_per_w)], idx_d, sem_id)
    cp_is.wait()
    cp_id.wait()

    sems = ((sem_s0, sem_d0), (sem_s1, sem_d1))

    def issue(ci, buf):
        ss, sd = sems[buf]
        sub = CHUNK // 5
        for j in range(5):
            pltpu.async_copy(
                z_hbm.at[idx_s.at[pl.ds(ci * CHUNK + j * sub, sub)]],
                rows_s.at[buf].at[pl.ds(j * sub, sub)], ss)
            pltpu.async_copy(
                z_hbm.at[idx_d.at[pl.ds(ci * CHUNK + j * sub, sub)]],
                rows_d.at[buf].at[pl.ds(j * sub, sub)], sd)

    def wait(buf):
        ss, sd = sems[buf]
        pltpu.make_async_copy(z_hbm.at[idx_s.at[pl.ds(0, CHUNK)]],
                              rows_s.at[buf], ss).wait()
        pltpu.make_async_copy(z_hbm.at[idx_d.at[pl.ds(0, CHUNK)]],
                              rows_d.at[buf], sd).wait()

    lane = lax.iota(jnp.int32, L)

    def compute(ci, buf):
        rs = rows_s.at[buf]
        rd = rows_d.at[buf]

        def group_wrap(g, carry2):
            def edge_body(em, tot):
                e = g * L + em
                acc = jnp.zeros((L,), jnp.float32)
                for k in range(0, w // L, 2):
                    vs0 = plsc.bitcast(rs[e, pl.ds(k * L, L)], jnp.bfloat16)
                    vd0 = plsc.bitcast(rd[e, pl.ds(k * L, L)], jnp.bfloat16)
                    vs1 = plsc.bitcast(rs[e, pl.ds((k + 1) * L, L)], jnp.bfloat16)
                    vd1 = plsc.bitcast(rd[e, pl.ds((k + 1) * L, L)], jnp.bfloat16)
                    p = vs0 * vd0 + vs1 * vd1
                    u0, u1 = plsc.unpack(p, format=plsc.PackFormat.INTERLEAVED)
                    acc = acc + u0 + u1
                val = jnp.sum(acc)
                return jnp.where(lane == em, jnp.full((L,), val, jnp.float32),
                                 tot)

            tot = lax.fori_loop(0, L, edge_body, jnp.zeros((L,), jnp.float32),
                                unroll=4)
            out_v[pl.ds(ci * CHUNK + g * L, L)] = tot
            return carry2

        lax.fori_loop(0, CHUNK // L, group_wrap, 0)

    issue(0, 0)

    def pair_body(h, carry):
        i = h * 2
        issue(i + 1, 1)
        wait(0)
        compute(i, 0)
        issue(i + 2, 0)
        wait(1)
        compute(i + 1, 1)
        return carry

    lax.fori_loop(0, (n_chunks - 1) // 2, pair_body, 0)
    wait(0)
    compute(n_chunks - 1, 0)

    pltpu.sync_copy(out_v, out_hbm.at[pl.ds(base, b_per_w)])


def kernel(z, source, destination):
    b = source.shape[0]
    d = z.shape[1]
    b_per_w = b // NW
    u = lax.bitcast_convert_type(z, jnp.uint32)
    r = (u + jnp.uint32(0x7FFF) + ((u >> 16) & jnp.uint32(1))) >> 16
    z = lax.bitcast_convert_type(r[:, :d // 2] | (r[:, d // 2:] << 16),
                                 jnp.int32)
    source = source.astype(jnp.int32)
    destination = destination.astype(jnp.int32)

    run = functools.partial(
        pl.kernel,
        mesh=plsc.VectorSubcoreMesh(core_axis_name="c", subcore_axis_name="s"),
        compiler_params=pltpu.CompilerParams(
            needs_layout_passes=False, use_tc_tiling_on_sc=False),
        out_type=jax.ShapeDtypeStruct((b,), jnp.float32),
        scratch_types=[
            pltpu.VMEM((b_per_w,), jnp.int32),
            pltpu.VMEM((b_per_w,), jnp.int32),
            pltpu.VMEM((2, CHUNK, d // 2), jnp.int32),
            pltpu.VMEM((2, CHUNK, d // 2), jnp.int32),
            pltpu.VMEM((b_per_w,), jnp.float32),
            pltpu.VMEM((L * L,), jnp.float32),
            pltpu.SemaphoreType.DMA,
            pltpu.SemaphoreType.DMA,
            pltpu.SemaphoreType.DMA,
            pltpu.SemaphoreType.DMA,
            pltpu.SemaphoreType.DMA,
            pltpu.SemaphoreType.DMA,
        ],
    )(_decoder_kernel)
    return run(z, source, destination)
